# Initial kernel scaffold; baseline (speedup 1.0000x reference)
import jax
import jax.numpy as jnp
from jax.experimental import pallas as pl


def kernel(X, edge_index, conv_w, conv_b, cheb_W, cheb_b, bn_gamma, bn_beta):
    raise NotImplementedError("write your pallas kernel here")


# speedup vs baseline: 6.9070x; 6.9070x over previous
import functools

import jax
import jax.numpy as jnp
from jax import lax
from jax.experimental import pallas as pl
from jax.experimental.pallas import tpu as pltpu
from jax.experimental.pallas import tpu_sc as plsc

T = 8192
N = 64
E = 131072
NCHUNK = 32
CW = 128
ACC_ROWS = 8448
ZROWS = 528
TRASH = 8192
NTILES = 16
EB = 128
NBATCH = E // NTILES // EB
TBLK = 128
NTBLK = T // TBLK

_mesh = plsc.VectorSubcoreMesh(core_axis_name="c", subcore_axis_name="s")


def _prep_body(src_ref, dst_ref, sabs_ref, srcm_ref, dstm_ref):
    c = pl.program_id(0)
    s = src_ref[...]
    d = dst_ref[...]
    sabs_ref[...] = (s + c * T)[None]

    @pl.when(c == 0)
    def _():
        loop = s == d
        srcm_ref[...] = jnp.where(loop, TRASH, s)
        dstm_ref[...] = jnp.where(loop, TRASH, d)


def _prep(src, dst):
    return pl.pallas_call(
        _prep_body,
        grid=(NCHUNK,),
        in_specs=[
            pl.BlockSpec((1, E), lambda c: (0, 0)),
            pl.BlockSpec((1, E), lambda c: (0, 0)),
        ],
        out_specs=[
            pl.BlockSpec((1, 1, E), lambda c: (c, 0, 0)),
            pl.BlockSpec((1, E), lambda c: (0, 0)),
            pl.BlockSpec((1, E), lambda c: (0, 0)),
        ],
        out_shape=[
            jax.ShapeDtypeStruct((NCHUNK, 1, E), jnp.int32),
            jax.ShapeDtypeStruct((1, E), jnp.int32),
            jax.ShapeDtypeStruct((1, E), jnp.int32),
        ],
    )(src, dst)


@functools.partial(
    pl.kernel,
    out_type=jax.ShapeDtypeStruct((2, T), jnp.float32),
    mesh=_mesh,
    scratch_types=[
        pltpu.VMEM((E // 32 // EB, EB), jnp.int32),
        pltpu.VMEM((EB,), jnp.float32),
        pltpu.VMEM((ZROWS,), jnp.float32),
        pltpu.VMEM_SHARED((ACC_ROWS,), jnp.float32),
    ],
)
def _deg_kernel(srcm_hbm, degp_hbm, idx_v, ones_v, zero_v, acc_sh):
    cid = lax.axis_index("c")
    sid = lax.axis_index("s")
    wid = cid * NTILES + sid
    one = jnp.ones((16,), jnp.float32)
    zero = jnp.zeros((16,), jnp.float32)
    for k in range(EB // 16):
        ones_v[pl.ds(k * 16, 16)] = one
    for k in range(ZROWS // 16):
        zero_v[pl.ds(k * 16, 16)] = zero
    pltpu.sync_copy(zero_v, acc_sh.at[pl.ds(sid * ZROWS, ZROWS)])
    pltpu.sync_copy(srcm_hbm.at[wid], idx_v)
    plsc.subcore_barrier()

    def body(j, carry):
        pltpu.sync_copy(ones_v, acc_sh.at[idx_v.at[j]], add=True)
        return carry

    lax.fori_loop(0, E // 32 // EB, body, 0)
    plsc.subcore_barrier()
    pltpu.sync_copy(
        acc_sh.at[pl.ds(sid * (T // NTILES), T // NTILES)],
        degp_hbm.at[cid, pl.ds(sid * (T // NTILES), T // NTILES)],
    )


def _conv_body(x_ref, w_ref, b_ref, degp_ref, y0_ref, u0_ref, dis_ref):
    x = x_ref[...]
    w = w_ref[...]
    z = jnp.dot(x.reshape(TBLK * N, N), w.T, preferred_element_type=jnp.float32)
    z = jnp.maximum(z + b_ref[...], 0.0)
    y = jnp.swapaxes(z.reshape(TBLK, N, N), 1, 2)
    y0_ref[...] = y
    dg = degp_ref[...]
    deg = dg[0] + dg[1]
    dis = jnp.where(deg > 0.0, lax.rsqrt(jnp.maximum(deg, 1e-12)), 0.0)
    dis_ref[...] = dis[None, :]
    u = y * dis[:, None, None]
    u0_ref[...] = jnp.swapaxes(u.reshape(TBLK, NCHUNK, 2, CW // 2), 0, 1)


def _conv(x3, w, b, degp):
    return pl.pallas_call(
        _conv_body,
        grid=(NTBLK,),
        in_specs=[
            pl.BlockSpec((TBLK, N, N), lambda i: (i, 0, 0)),
            pl.BlockSpec((N, N), lambda i: (0, 0)),
            pl.BlockSpec((1, N), lambda i: (0, 0)),
            pl.BlockSpec((2, TBLK), lambda i: (0, i)),
        ],
        out_specs=[
            pl.BlockSpec((TBLK, N, N), lambda i: (i, 0, 0)),
            pl.BlockSpec((NCHUNK, TBLK, 2, CW // 2), lambda i: (0, i, 0, 0)),
            pl.BlockSpec((1, TBLK), lambda i: (0, i)),
        ],
        out_shape=[
            jax.ShapeDtypeStruct((T, N, N), jnp.float32),
            jax.ShapeDtypeStruct((NCHUNK, T, 2, CW // 2), jnp.float32),
            jax.ShapeDtypeStruct((1, T), jnp.float32),
        ],
    )(x3, w, b, degp)


@functools.partial(
    pl.kernel,
    out_type=jax.ShapeDtypeStruct((NCHUNK * T, CW), jnp.float32),
    mesh=_mesh,
    scratch_types=[
        pltpu.VMEM((NBATCH, EB), jnp.int32),
        pltpu.VMEM((NBATCH, EB), jnp.int32),
        pltpu.VMEM((EB, CW), jnp.float32),
        pltpu.VMEM((ZROWS // 3, CW), jnp.float32),
        pltpu.VMEM_SHARED((ACC_ROWS, CW), jnp.float32),
        pltpu.SemaphoreType.DMA,
    ],
)
def _spmm_kernel(u_hbm, sabs_hbm, dstm_hbm, p_hbm,
                 sidx_v, didx_v, rows_v, zero_v, acc_sh, sem):
    cid = lax.axis_index("c")
    sid = lax.axis_index("s")
    zero = jnp.zeros((16,), jnp.float32)

    def zb(i, carry):
        for k in range(CW // 16):
            zero_v[i, pl.ds(k * 16, 16)] = zero
        return carry

    lax.fori_loop(0, ZROWS // 3, zb, 0)
    pltpu.sync_copy(dstm_hbm.at[sid], didx_v)

    def chunk_body(k, carry):
        c = cid * NTILES + k
        for q in range(3):
            pltpu.sync_copy(
                zero_v,
                acc_sh.at[pl.ds(sid * ZROWS + q * (ZROWS // 3), ZROWS // 3)],
            )
        pltpu.sync_copy(sabs_hbm.at[c, sid], sidx_v)
        plsc.subcore_barrier()

        def batch_body(j, bcarry):
            pltpu.async_copy(u_hbm.at[sidx_v.at[j]], rows_v, sem).wait()
            pltpu.sync_copy(rows_v, acc_sh.at[didx_v.at[j]], add=True)
            return bcarry

        lax.fori_loop(0, NBATCH, batch_body, 0)
        plsc.subcore_barrier()
        pltpu.sync_copy(
            acc_sh.at[pl.ds(sid * (T // NTILES), T // NTILES)],
            p_hbm.at[pl.ds(c * T + sid * (T // NTILES), T // NTILES)],
        )
        plsc.subcore_barrier()
        return carry

    lax.fori_loop(0, NTILES, chunk_body, 0)


def _scale_body(p_ref, dis_ref, u_ref):
    d = dis_ref[...]
    u_ref[...] = -(d * d)[:, :, None] * p_ref[...]


def _scale(p_cm, dis):
    return pl.pallas_call(
        _scale_body,
        grid=(NCHUNK, NTBLK),
        in_specs=[
            pl.BlockSpec((1, TBLK, CW), lambda c, i: (c, i, 0)),
            pl.BlockSpec((1, TBLK), lambda c, i: (0, i)),
        ],
        out_specs=pl.BlockSpec((1, TBLK, CW), lambda c, i: (c, i, 0)),
        out_shape=jax.ShapeDtypeStruct((NCHUNK, T, CW), jnp.float32),
    )(p_cm, dis)


def _cheb_body(y0_ref, p1_ref, p2_ref, dis_ref, w_ref, b_ref, r_ref, st_ref):
    y0 = y0_ref[...]
    p1 = jnp.swapaxes(p1_ref[...], 0, 1).reshape(TBLK, N, N)
    p2 = jnp.swapaxes(p2_ref[...], 0, 1).reshape(TBLK, N, N)
    w = w_ref[...]
    m1 = jnp.dot(y0.reshape(-1, N), w[0] - w[2],
                 preferred_element_type=jnp.float32)
    m2 = (jnp.dot(p1.reshape(-1, N), w[1], preferred_element_type=jnp.float32)
          + jnp.dot(p2.reshape(-1, N), 2.0 * w[2],
                    preferred_element_type=jnp.float32))
    d = dis_ref[...][0]
    r = (m1.reshape(TBLK, N, N)
         - d[:, None, None] * m2.reshape(TBLK, N, N)
         + b_ref[...][0][None, None, :])
    r_ref[...] = r
    s1 = r.sum(axis=2).sum(axis=0)
    s2 = (r * r).sum(axis=2).sum(axis=0)
    st_ref[...] = jnp.concatenate([s1[None, :], s2[None, :]], axis=0)[None]


def _cheb(y0, p1_cm4, p2_cm4, dis, cheb_W, cheb_b2):
    return pl.pallas_call(
        _cheb_body,
        grid=(NTBLK,),
        in_specs=[
            pl.BlockSpec((TBLK, N, N), lambda i: (i, 0, 0)),
            pl.BlockSpec((NCHUNK, TBLK, 2, CW // 2), lambda i: (0, i, 0, 0)),
            pl.BlockSpec((NCHUNK, TBLK, 2, CW // 2), lambda i: (0, i, 0, 0)),
            pl.BlockSpec((1, TBLK), lambda i: (0, i)),
            pl.BlockSpec((3, N, N), lambda i: (0, 0, 0)),
            pl.BlockSpec((1, N), lambda i: (0, 0)),
        ],
        out_specs=[
            pl.BlockSpec((TBLK, N, N), lambda i: (i, 0, 0)),
            pl.BlockSpec((1, 2, N), lambda i: (i, 0, 0)),
        ],
        out_shape=[
            jax.ShapeDtypeStruct((T, N, N), jnp.float32),
            jax.ShapeDtypeStruct((NTBLK, 2, N), jnp.float32),
        ],
    )(y0, p1_cm4, p2_cm4, dis, cheb_W, cheb_b2)


def _bn_body(r_ref, x_ref, st_ref, g_ref, be_ref, out_ref):
    st = st_ref[...].sum(axis=0)
    cnt = float(T * N)
    mu = st[0] / cnt
    var = st[1] / cnt - mu * mu
    scale = g_ref[...][0] * lax.rsqrt(var + 1e-5)
    shift = be_ref[...][0] - mu * scale
    out = r_ref[...] * scale[None, :, None] + shift[None, :, None] + x_ref[...]
    out_ref[...] = jnp.maximum(out, 0.0)


def _bn(r, x3, stats, g2, b2):
    return pl.pallas_call(
        _bn_body,
        grid=(NTBLK,),
        in_specs=[
            pl.BlockSpec((TBLK, N, N), lambda i: (i, 0, 0)),
            pl.BlockSpec((TBLK, N, N), lambda i: (i, 0, 0)),
            pl.BlockSpec((NTBLK, 2, N), lambda i: (0, 0, 0)),
            pl.BlockSpec((1, N), lambda i: (0, 0)),
            pl.BlockSpec((1, N), lambda i: (0, 0)),
        ],
        out_specs=pl.BlockSpec((TBLK, N, N), lambda i: (i, 0, 0)),
        out_shape=jax.ShapeDtypeStruct((T, N, N), jnp.float32),
    )(r, x3, stats, g2, b2)


def kernel(X, edge_index, conv_w, conv_b, cheb_W, cheb_b, bn_gamma, bn_beta):
    x3 = X[0]
    src = edge_index[0].reshape(1, E)
    dst = edge_index[1].reshape(1, E)

    sabs, srcm, dstm = _prep(src, dst)
    srcm_w = srcm.reshape(32, E // 32 // EB, EB)
    dstm3 = dstm.reshape(NTILES, NBATCH, EB)
    sabs4 = sabs.reshape(NCHUNK, NTILES, NBATCH, EB)

    degp = _deg_kernel(srcm_w)
    y0, u0_cm4, dis = _conv(x3, conv_w[:, :, 0, 0], conv_b.reshape(1, N), degp)

    p1 = _spmm_kernel(u0_cm4.reshape(NCHUNK * T, CW), sabs4, dstm3)
    u1 = _scale(p1.reshape(NCHUNK, T, CW), dis)
    p2 = _spmm_kernel(u1.reshape(NCHUNK * T, CW), sabs4, dstm3)
    p1_cm4 = p1.reshape(NCHUNK, T, 2, CW // 2)
    p2_cm4 = p2.reshape(NCHUNK, T, 2, CW // 2)

    r, stats = _cheb(y0, p1_cm4, p2_cm4, dis, cheb_W, cheb_b.reshape(1, N))
    out = _bn(r, x3, stats, bn_gamma.reshape(1, N), bn_beta.reshape(1, N))
    return out[None]


# speedup vs baseline: 8.0808x; 1.1699x over previous
import functools

import jax
import jax.numpy as jnp
from jax import lax
from jax.experimental import pallas as pl
from jax.experimental.pallas import tpu as pltpu
from jax.experimental.pallas import tpu_sc as plsc

T = 8192
N = 64
E = 131072
NCHUNK = 32
CW = 128
ACC_ROWS = 8448
ZROWS = 528
TRASH = 8192
NTILES = 16
EB = 128
NBATCH = E // NTILES // EB
TBLK = 128
NTBLK = T // TBLK

_mesh = plsc.VectorSubcoreMesh(core_axis_name="c", subcore_axis_name="s")


def _prep_body(src_ref, dst_ref, sabs_ref, srcm_ref, dstm_ref):
    c = pl.program_id(0)
    s = src_ref[...]
    d = dst_ref[...]
    sabs_ref[...] = (s + c * T)[None]

    @pl.when(c == 0)
    def _():
        loop = s == d
        srcm_ref[...] = jnp.where(loop, TRASH, s)
        dstm_ref[...] = jnp.where(loop, TRASH, d)


def _prep(src, dst):
    return pl.pallas_call(
        _prep_body,
        grid=(NCHUNK,),
        in_specs=[
            pl.BlockSpec((1, E), lambda c: (0, 0)),
            pl.BlockSpec((1, E), lambda c: (0, 0)),
        ],
        out_specs=[
            pl.BlockSpec((1, 1, E), lambda c: (c, 0, 0)),
            pl.BlockSpec((1, E), lambda c: (0, 0)),
            pl.BlockSpec((1, E), lambda c: (0, 0)),
        ],
        out_shape=[
            jax.ShapeDtypeStruct((NCHUNK, 1, E), jnp.int32),
            jax.ShapeDtypeStruct((1, E), jnp.int32),
            jax.ShapeDtypeStruct((1, E), jnp.int32),
        ],
    )(src, dst)


@functools.partial(
    pl.kernel,
    out_type=jax.ShapeDtypeStruct((2, T), jnp.float32),
    mesh=_mesh,
    scratch_types=[
        pltpu.VMEM((E // 32 // EB, EB), jnp.int32),
        pltpu.VMEM((EB,), jnp.float32),
        pltpu.VMEM((ZROWS,), jnp.float32),
        pltpu.VMEM_SHARED((ACC_ROWS,), jnp.float32),
    ],
)
def _deg_kernel(srcm_hbm, degp_hbm, idx_v, ones_v, zero_v, acc_sh):
    cid = lax.axis_index("c")
    sid = lax.axis_index("s")
    wid = cid * NTILES + sid
    one = jnp.ones((16,), jnp.float32)
    zero = jnp.zeros((16,), jnp.float32)
    for k in range(EB // 16):
        ones_v[pl.ds(k * 16, 16)] = one
    for k in range(ZROWS // 16):
        zero_v[pl.ds(k * 16, 16)] = zero
    pltpu.sync_copy(zero_v, acc_sh.at[pl.ds(sid * ZROWS, ZROWS)])
    pltpu.sync_copy(srcm_hbm.at[wid], idx_v)
    plsc.subcore_barrier()

    def body(j, carry):
        pltpu.sync_copy(ones_v, acc_sh.at[idx_v.at[j]], add=True)
        return carry

    lax.fori_loop(0, E // 32 // EB, body, 0)
    plsc.subcore_barrier()
    pltpu.sync_copy(
        acc_sh.at[pl.ds(sid * (T // NTILES), T // NTILES)],
        degp_hbm.at[cid, pl.ds(sid * (T // NTILES), T // NTILES)],
    )


def _conv_body(x_ref, w_ref, b_ref, degp_ref, y0_ref, u0_ref, dis_ref):
    x = x_ref[...]
    w = w_ref[...]
    z = jnp.dot(x.reshape(TBLK * N, N), w.T, preferred_element_type=jnp.float32)
    z = jnp.maximum(z + b_ref[...], 0.0)
    y = jnp.swapaxes(z.reshape(TBLK, N, N), 1, 2)
    y0_ref[...] = y
    dg = degp_ref[...]
    deg = dg[0] + dg[1]
    dis = jnp.where(deg > 0.0, lax.rsqrt(jnp.maximum(deg, 1e-12)), 0.0)
    dis_ref[...] = dis[None, :]
    u = y * dis[:, None, None]
    u0_ref[...] = jnp.swapaxes(u.reshape(TBLK, NCHUNK, 2, CW // 2), 0, 1)


def _conv(x3, w, b, degp):
    return pl.pallas_call(
        _conv_body,
        grid=(NTBLK,),
        in_specs=[
            pl.BlockSpec((TBLK, N, N), lambda i: (i, 0, 0)),
            pl.BlockSpec((N, N), lambda i: (0, 0)),
            pl.BlockSpec((1, N), lambda i: (0, 0)),
            pl.BlockSpec((2, TBLK), lambda i: (0, i)),
        ],
        out_specs=[
            pl.BlockSpec((TBLK, N, N), lambda i: (i, 0, 0)),
            pl.BlockSpec((NCHUNK, TBLK, 2, CW // 2), lambda i: (0, i, 0, 0)),
            pl.BlockSpec((1, TBLK), lambda i: (0, i)),
        ],
        out_shape=[
            jax.ShapeDtypeStruct((T, N, N), jnp.float32),
            jax.ShapeDtypeStruct((NCHUNK, T, 2, CW // 2), jnp.float32),
            jax.ShapeDtypeStruct((1, T), jnp.float32),
        ],
    )(x3, w, b, degp)


@functools.partial(
    pl.kernel,
    out_type=jax.ShapeDtypeStruct((NCHUNK * T, CW), jnp.float32),
    mesh=_mesh,
    scratch_types=[
        pltpu.VMEM((NBATCH, EB), jnp.int32),
        pltpu.VMEM((NBATCH, EB), jnp.int32),
        pltpu.VMEM((EB, CW), jnp.float32),
        pltpu.VMEM((EB, CW), jnp.float32),
        pltpu.VMEM((ZROWS // 6, CW), jnp.float32),
        pltpu.VMEM_SHARED((ACC_ROWS, CW), jnp.float32),
        pltpu.SemaphoreType.DMA,
    ],
)
def _spmm_kernel(u_hbm, sabs_hbm, dstm_hbm, p_hbm,
                 sidx_v, didx_v, rows_a, rows_b, zero_v, acc_sh, sem):
    cid = lax.axis_index("c")
    sid = lax.axis_index("s")
    zero = jnp.zeros((16,), jnp.float32)

    def zb(i, carry):
        for k in range(CW // 16):
            zero_v[i, pl.ds(k * 16, 16)] = zero
        return carry

    lax.fori_loop(0, ZROWS // 6, zb, 0)
    pltpu.sync_copy(dstm_hbm.at[sid], didx_v)

    def chunk_body(k, carry):
        c = cid * NTILES + k
        for q in range(6):
            pltpu.sync_copy(
                zero_v,
                acc_sh.at[pl.ds(sid * ZROWS + q * (ZROWS // 6), ZROWS // 6)],
            )
        pltpu.sync_copy(sabs_hbm.at[c, sid], sidx_v)
        plsc.subcore_barrier()

        pltpu.async_copy(u_hbm.at[sidx_v.at[0]], rows_a, sem)

        def batch_body(i, bcarry):
            ja = 2 * i
            jb = 2 * i + 1
            pltpu.make_async_copy(u_hbm.at[sidx_v.at[0]], rows_a, sem).wait()
            pltpu.async_copy(u_hbm.at[sidx_v.at[jb]], rows_b, sem)
            pltpu.sync_copy(rows_a, acc_sh.at[didx_v.at[ja]], add=True)
            pltpu.make_async_copy(u_hbm.at[sidx_v.at[0]], rows_b, sem).wait()
            nxt = jnp.minimum(jb + 1, NBATCH - 1)
            pltpu.async_copy(u_hbm.at[sidx_v.at[nxt]], rows_a, sem)
            pltpu.sync_copy(rows_b, acc_sh.at[didx_v.at[jb]], add=True)
            return bcarry

        lax.fori_loop(0, NBATCH // 2, batch_body, 0)
        pltpu.make_async_copy(u_hbm.at[sidx_v.at[0]], rows_a, sem).wait()
        plsc.subcore_barrier()
        pltpu.sync_copy(
            acc_sh.at[pl.ds(sid * (T // NTILES), T // NTILES)],
            p_hbm.at[pl.ds(c * T + sid * (T // NTILES), T // NTILES)],
        )
        plsc.subcore_barrier()
        return carry

    lax.fori_loop(0, NTILES, chunk_body, 0)


def _scale_body(p_ref, dis_ref, u_ref):
    d = dis_ref[...]
    u_ref[...] = -(d * d)[:, :, None] * p_ref[...]


def _scale(p_cm, dis):
    return pl.pallas_call(
        _scale_body,
        grid=(NCHUNK, NTBLK),
        in_specs=[
            pl.BlockSpec((1, TBLK, CW), lambda c, i: (c, i, 0)),
            pl.BlockSpec((1, TBLK), lambda c, i: (0, i)),
        ],
        out_specs=pl.BlockSpec((1, TBLK, CW), lambda c, i: (c, i, 0)),
        out_shape=jax.ShapeDtypeStruct((NCHUNK, T, CW), jnp.float32),
    )(p_cm, dis)


def _cheb_body(y0_ref, p1_ref, p2_ref, dis_ref, w_ref, b_ref, r_ref, st_ref):
    y0 = y0_ref[...]
    p1 = jnp.swapaxes(p1_ref[...], 0, 1).reshape(TBLK, N, N)
    p2 = jnp.swapaxes(p2_ref[...], 0, 1).reshape(TBLK, N, N)
    w = w_ref[...]
    m1 = jnp.dot(y0.reshape(-1, N), w[0] - w[2],
                 preferred_element_type=jnp.float32)
    m2 = (jnp.dot(p1.reshape(-1, N), w[1], preferred_element_type=jnp.float32)
          + jnp.dot(p2.reshape(-1, N), 2.0 * w[2],
                    preferred_element_type=jnp.float32))
    d = dis_ref[...][0]
    r = (m1.reshape(TBLK, N, N)
         - d[:, None, None] * m2.reshape(TBLK, N, N)
         + b_ref[...][0][None, None, :])
    r_ref[...] = r
    s1 = r.sum(axis=2).sum(axis=0)
    s2 = (r * r).sum(axis=2).sum(axis=0)
    st_ref[...] = jnp.concatenate([s1[None, :], s2[None, :]], axis=0)[None]


def _cheb(y0, p1_cm4, p2_cm4, dis, cheb_W, cheb_b2):
    return pl.pallas_call(
        _cheb_body,
        grid=(NTBLK,),
        in_specs=[
            pl.BlockSpec((TBLK, N, N), lambda i: (i, 0, 0)),
            pl.BlockSpec((NCHUNK, TBLK, 2, CW // 2), lambda i: (0, i, 0, 0)),
            pl.BlockSpec((NCHUNK, TBLK, 2, CW // 2), lambda i: (0, i, 0, 0)),
            pl.BlockSpec((1, TBLK), lambda i: (0, i)),
            pl.BlockSpec((3, N, N), lambda i: (0, 0, 0)),
            pl.BlockSpec((1, N), lambda i: (0, 0)),
        ],
        out_specs=[
            pl.BlockSpec((TBLK, N, N), lambda i: (i, 0, 0)),
            pl.BlockSpec((1, 2, N), lambda i: (i, 0, 0)),
        ],
        out_shape=[
            jax.ShapeDtypeStruct((T, N, N), jnp.float32),
            jax.ShapeDtypeStruct((NTBLK, 2, N), jnp.float32),
        ],
    )(y0, p1_cm4, p2_cm4, dis, cheb_W, cheb_b2)


def _bn_body(r_ref, x_ref, st_ref, g_ref, be_ref, out_ref):
    st = st_ref[...].sum(axis=0)
    cnt = float(T * N)
    mu = st[0] / cnt
    var = st[1] / cnt - mu * mu
    scale = g_ref[...][0] * lax.rsqrt(var + 1e-5)
    shift = be_ref[...][0] - mu * scale
    out = r_ref[...] * scale[None, :, None] + shift[None, :, None] + x_ref[...]
    out_ref[...] = jnp.maximum(out, 0.0)


def _bn(r, x3, stats, g2, b2):
    return pl.pallas_call(
        _bn_body,
        grid=(NTBLK,),
        in_specs=[
            pl.BlockSpec((TBLK, N, N), lambda i: (i, 0, 0)),
            pl.BlockSpec((TBLK, N, N), lambda i: (i, 0, 0)),
            pl.BlockSpec((NTBLK, 2, N), lambda i: (0, 0, 0)),
            pl.BlockSpec((1, N), lambda i: (0, 0)),
            pl.BlockSpec((1, N), lambda i: (0, 0)),
        ],
        out_specs=pl.BlockSpec((TBLK, N, N), lambda i: (i, 0, 0)),
        out_shape=jax.ShapeDtypeStruct((T, N, N), jnp.float32),
    )(r, x3, stats, g2, b2)


def kernel(X, edge_index, conv_w, conv_b, cheb_W, cheb_b, bn_gamma, bn_beta):
    x3 = X[0]
    src = edge_index[0].reshape(1, E)
    dst = edge_index[1].reshape(1, E)

    sabs, srcm, dstm = _prep(src, dst)
    srcm_w = srcm.reshape(32, E // 32 // EB, EB)
    dstm3 = dstm.reshape(NTILES, NBATCH, EB)
    sabs4 = sabs.reshape(NCHUNK, NTILES, NBATCH, EB)

    degp = _deg_kernel(srcm_w)
    y0, u0_cm4, dis = _conv(x3, conv_w[:, :, 0, 0], conv_b.reshape(1, N), degp)

    p1 = _spmm_kernel(u0_cm4.reshape(NCHUNK * T, CW), sabs4, dstm3)
    u1 = _scale(p1.reshape(NCHUNK, T, CW), dis)
    p2 = _spmm_kernel(u1.reshape(NCHUNK * T, CW), sabs4, dstm3)
    p1_cm4 = p1.reshape(NCHUNK, T, 2, CW // 2)
    p2_cm4 = p2.reshape(NCHUNK, T, 2, CW // 2)

    r, stats = _cheb(y0, p1_cm4, p2_cm4, dis, cheb_W, cheb_b.reshape(1, N))
    out = _bn(r, x3, stats, bn_gamma.reshape(1, N), bn_beta.reshape(1, N))
    return out[None]


# speedup vs baseline: 9.4588x; 1.1705x over previous
import functools

import jax
import jax.numpy as jnp
from jax import lax
from jax.experimental import pallas as pl
from jax.experimental.pallas import tpu as pltpu
from jax.experimental.pallas import tpu_sc as plsc

T = 8192
N = 64
E = 131072
NCHUNK = 32
CW = 128
ACC_ROWS = 8448
ZROWS = 528
TRASH = 8192
NTILES = 16
EB = 128
NBATCH = E // NTILES // EB
TBLK = 128
NTBLK = T // TBLK

_mesh = plsc.VectorSubcoreMesh(core_axis_name="c", subcore_axis_name="s")


def _prep_body(src_ref, dst_ref, sabs_ref, srcm_ref, dstm_ref):
    c = pl.program_id(0)
    s = src_ref[...]
    d = dst_ref[...]
    sabs_ref[...] = (s + c * T)[None]

    @pl.when(c == 0)
    def _():
        loop = s == d
        srcm_ref[...] = jnp.where(loop, TRASH, s)
        dstm_ref[...] = jnp.where(loop, TRASH, d)


def _prep(src, dst):
    return pl.pallas_call(
        _prep_body,
        grid=(NCHUNK,),
        in_specs=[
            pl.BlockSpec((1, E), lambda c: (0, 0)),
            pl.BlockSpec((1, E), lambda c: (0, 0)),
        ],
        out_specs=[
            pl.BlockSpec((1, 1, E), lambda c: (c, 0, 0)),
            pl.BlockSpec((1, E), lambda c: (0, 0)),
            pl.BlockSpec((1, E), lambda c: (0, 0)),
        ],
        out_shape=[
            jax.ShapeDtypeStruct((NCHUNK, 1, E), jnp.int32),
            jax.ShapeDtypeStruct((1, E), jnp.int32),
            jax.ShapeDtypeStruct((1, E), jnp.int32),
        ],
    )(src, dst)


@functools.partial(
    pl.kernel,
    out_type=jax.ShapeDtypeStruct((2, T), jnp.float32),
    mesh=_mesh,
    scratch_types=[
        pltpu.VMEM((E // 32 // EB, EB), jnp.int32),
        pltpu.VMEM((EB,), jnp.float32),
        pltpu.VMEM((ZROWS,), jnp.float32),
        pltpu.VMEM_SHARED((ACC_ROWS,), jnp.float32),
    ],
)
def _deg_kernel(srcm_hbm, degp_hbm, idx_v, ones_v, zero_v, acc_sh):
    cid = lax.axis_index("c")
    sid = lax.axis_index("s")
    wid = cid * NTILES + sid
    one = jnp.ones((16,), jnp.float32)
    zero = jnp.zeros((16,), jnp.float32)
    for k in range(EB // 16):
        ones_v[pl.ds(k * 16, 16)] = one
    for k in range(ZROWS // 16):
        zero_v[pl.ds(k * 16, 16)] = zero
    pltpu.sync_copy(zero_v, acc_sh.at[pl.ds(sid * ZROWS, ZROWS)])
    pltpu.sync_copy(srcm_hbm.at[wid], idx_v)
    plsc.subcore_barrier()

    def body(j, carry):
        pltpu.sync_copy(ones_v, acc_sh.at[idx_v.at[j]], add=True)
        return carry

    lax.fori_loop(0, E // 32 // EB, body, 0)
    plsc.subcore_barrier()
    pltpu.sync_copy(
        acc_sh.at[pl.ds(sid * (T // NTILES), T // NTILES)],
        degp_hbm.at[cid, pl.ds(sid * (T // NTILES), T // NTILES)],
    )


def _conv_body(x_ref, w_ref, b_ref, degp_ref, y0_ref, u0_ref, dis_ref):
    x = x_ref[...]
    w = w_ref[...]
    z = jnp.dot(x.reshape(TBLK * N, N), w.T, preferred_element_type=jnp.float32)
    z = jnp.maximum(z + b_ref[...], 0.0)
    y = jnp.swapaxes(z.reshape(TBLK, N, N), 1, 2)
    dg = degp_ref[...]
    deg = dg[0] + dg[1]
    dis = jnp.where(deg > 0.0, lax.rsqrt(jnp.maximum(deg, 1e-12)), 0.0)
    dis_ref[...] = dis[None, :]
    u = y * dis[:, None, None]
    for c in range(NCHUNK):
        y0_ref[c] = jnp.concatenate([y[:, 2 * c, :], y[:, 2 * c + 1, :]], axis=1)
        u0_ref[c] = jnp.concatenate([u[:, 2 * c, :], u[:, 2 * c + 1, :]], axis=1)


def _conv(x3, w, b, degp):
    return pl.pallas_call(
        _conv_body,
        grid=(NTBLK,),
        in_specs=[
            pl.BlockSpec((TBLK, N, N), lambda i: (i, 0, 0)),
            pl.BlockSpec((N, N), lambda i: (0, 0)),
            pl.BlockSpec((1, N), lambda i: (0, 0)),
            pl.BlockSpec((2, TBLK), lambda i: (0, i)),
        ],
        out_specs=[
            pl.BlockSpec((NCHUNK, TBLK, CW), lambda i: (0, i, 0)),
            pl.BlockSpec((NCHUNK, TBLK, CW), lambda i: (0, i, 0)),
            pl.BlockSpec((1, TBLK), lambda i: (0, i)),
        ],
        out_shape=[
            jax.ShapeDtypeStruct((NCHUNK, T, CW), jnp.float32),
            jax.ShapeDtypeStruct((NCHUNK, T, CW), jnp.float32),
            jax.ShapeDtypeStruct((1, T), jnp.float32),
        ],
    )(x3, w, b, degp)


@functools.partial(
    pl.kernel,
    out_type=jax.ShapeDtypeStruct((NCHUNK * T, CW), jnp.float32),
    mesh=_mesh,
    scratch_types=[
        pltpu.VMEM((NBATCH, EB), jnp.int32),
        pltpu.VMEM((NBATCH, EB), jnp.int32),
        pltpu.VMEM((EB, CW), jnp.float32),
        pltpu.VMEM((EB, CW), jnp.float32),
        pltpu.VMEM((ZROWS // 6, CW), jnp.float32),
        pltpu.VMEM_SHARED((ACC_ROWS, CW), jnp.float32),
        pltpu.SemaphoreType.DMA,
    ],
)
def _spmm_kernel(u_hbm, sabs_hbm, dstm_hbm, p_hbm,
                 sidx_v, didx_v, rows_a, rows_b, zero_v, acc_sh, sem):
    cid = lax.axis_index("c")
    sid = lax.axis_index("s")
    zero = jnp.zeros((16,), jnp.float32)

    def zb(i, carry):
        for k in range(CW // 16):
            zero_v[i, pl.ds(k * 16, 16)] = zero
        return carry

    lax.fori_loop(0, ZROWS // 6, zb, 0)
    pltpu.sync_copy(dstm_hbm.at[sid], didx_v)

    def chunk_body(k, carry):
        c = cid * NTILES + k
        for q in range(6):
            pltpu.sync_copy(
                zero_v,
                acc_sh.at[pl.ds(sid * ZROWS + q * (ZROWS // 6), ZROWS // 6)],
            )
        pltpu.sync_copy(sabs_hbm.at[c, sid], sidx_v)
        plsc.subcore_barrier()

        pltpu.async_copy(u_hbm.at[sidx_v.at[0]], rows_a, sem)

        def batch_body(i, bcarry):
            ja = 2 * i
            jb = 2 * i + 1
            pltpu.make_async_copy(u_hbm.at[sidx_v.at[0]], rows_a, sem).wait()
            pltpu.async_copy(u_hbm.at[sidx_v.at[jb]], rows_b, sem)
            pltpu.sync_copy(rows_a, acc_sh.at[didx_v.at[ja]], add=True)
            pltpu.make_async_copy(u_hbm.at[sidx_v.at[0]], rows_b, sem).wait()
            nxt = jnp.minimum(jb + 1, NBATCH - 1)
            pltpu.async_copy(u_hbm.at[sidx_v.at[nxt]], rows_a, sem)
            pltpu.sync_copy(rows_b, acc_sh.at[didx_v.at[jb]], add=True)
            return bcarry

        lax.fori_loop(0, NBATCH // 2, batch_body, 0)
        pltpu.make_async_copy(u_hbm.at[sidx_v.at[0]], rows_a, sem).wait()
        plsc.subcore_barrier()
        pltpu.sync_copy(
            acc_sh.at[pl.ds(sid * (T // NTILES), T // NTILES)],
            p_hbm.at[pl.ds(c * T + sid * (T // NTILES), T // NTILES)],
        )
        plsc.subcore_barrier()
        return carry

    lax.fori_loop(0, NTILES, chunk_body, 0)


def _scale_body(p_ref, dis_ref, u_ref):
    d = dis_ref[...]
    u_ref[...] = -(d * d)[:, :, None] * p_ref[...]


def _scale(p_cm, dis):
    return pl.pallas_call(
        _scale_body,
        grid=(NCHUNK, NTBLK),
        in_specs=[
            pl.BlockSpec((1, TBLK, CW), lambda c, i: (c, i, 0)),
            pl.BlockSpec((1, TBLK), lambda c, i: (0, i)),
        ],
        out_specs=pl.BlockSpec((1, TBLK, CW), lambda c, i: (c, i, 0)),
        out_shape=jax.ShapeDtypeStruct((NCHUNK, T, CW), jnp.float32),
    )(p_cm, dis)


def _bd(w):
    zz = jnp.zeros((N, N), jnp.float32)
    top = jnp.concatenate([w, zz], axis=1)
    bot = jnp.concatenate([zz, w], axis=1)
    return jnp.concatenate([top, bot], axis=0)


def _cheb_body(y0_ref, p1_ref, p2_ref, dis_ref, w_ref, b_ref, r_ref, st_ref):
    y0 = y0_ref[...].reshape(NCHUNK * TBLK, CW)
    p1 = p1_ref[...].reshape(NCHUNK * TBLK, CW)
    p2 = p2_ref[...].reshape(NCHUNK * TBLK, CW)
    w = w_ref[...]
    m1 = jnp.dot(y0, _bd(w[0] - w[2]), preferred_element_type=jnp.float32)
    m2 = (jnp.dot(p1, _bd(w[1]), preferred_element_type=jnp.float32)
          + jnp.dot(p2, _bd(2.0 * w[2]), preferred_element_type=jnp.float32))
    d = dis_ref[...][0]
    r = (m1.reshape(NCHUNK, TBLK, CW)
         - d[None, :, None] * m2.reshape(NCHUNK, TBLK, CW)
         + b_ref[...][0][None, None, :])
    r_ref[...] = r
    s1 = r.sum(axis=1)
    s2 = (r * r).sum(axis=1)
    st_ref[...] = jnp.concatenate([s1, s2], axis=0)[None]


def _cheb(y0_cm, p1_cm, p2_cm, dis, cheb_W, cheb_b2):
    return pl.pallas_call(
        _cheb_body,
        grid=(NTBLK,),
        in_specs=[
            pl.BlockSpec((NCHUNK, TBLK, CW), lambda i: (0, i, 0)),
            pl.BlockSpec((NCHUNK, TBLK, CW), lambda i: (0, i, 0)),
            pl.BlockSpec((NCHUNK, TBLK, CW), lambda i: (0, i, 0)),
            pl.BlockSpec((1, TBLK), lambda i: (0, i)),
            pl.BlockSpec((3, N, N), lambda i: (0, 0, 0)),
            pl.BlockSpec((1, CW), lambda i: (0, 0)),
        ],
        out_specs=[
            pl.BlockSpec((NCHUNK, TBLK, CW), lambda i: (0, i, 0)),
            pl.BlockSpec((1, 2 * NCHUNK, CW), lambda i: (i, 0, 0)),
        ],
        out_shape=[
            jax.ShapeDtypeStruct((NCHUNK, T, CW), jnp.float32),
            jax.ShapeDtypeStruct((NTBLK, 2 * NCHUNK, CW), jnp.float32),
        ],
    )(y0_cm, p1_cm, p2_cm, dis, cheb_W, cheb_b2)


def _bn_body(r_ref, x_ref, st_ref, g_ref, be_ref, out_ref):
    st = st_ref[...].sum(axis=0)
    cnt = float(T * N)
    s1l = st[:NCHUNK, : N].sum(axis=1)[:, None]
    s1r = st[:NCHUNK, N:].sum(axis=1)[:, None]
    s2l = st[NCHUNK:, : N].sum(axis=1)[:, None]
    s2r = st[NCHUNK:, N:].sum(axis=1)[:, None]
    ones = jnp.ones((1, N), jnp.float32)
    mu = jnp.concatenate([s1l * ones, s1r * ones], axis=1) / cnt
    s2 = jnp.concatenate([s2l * ones, s2r * ones], axis=1) / cnt
    var = s2 - mu * mu
    scale = g_ref[...] * lax.rsqrt(var + 1e-5)
    shift = be_ref[...] - mu * scale
    q = r_ref[...] * scale[:, None, :] + shift[:, None, :]
    x = x_ref[...]
    for c in range(NCHUNK):
        out_ref[:, 2 * c, :] = jnp.maximum(q[c, :, :N] + x[:, 2 * c, :], 0.0)
        out_ref[:, 2 * c + 1, :] = jnp.maximum(q[c, :, N:] + x[:, 2 * c + 1, :], 0.0)


def _bn(r_cm, x3, stats, gp, bp):
    return pl.pallas_call(
        _bn_body,
        grid=(NTBLK,),
        in_specs=[
            pl.BlockSpec((NCHUNK, TBLK, CW), lambda i: (0, i, 0)),
            pl.BlockSpec((TBLK, N, N), lambda i: (i, 0, 0)),
            pl.BlockSpec((NTBLK, 2 * NCHUNK, CW), lambda i: (0, 0, 0)),
            pl.BlockSpec((NCHUNK, CW), lambda i: (0, 0)),
            pl.BlockSpec((NCHUNK, CW), lambda i: (0, 0)),
        ],
        out_specs=pl.BlockSpec((TBLK, N, N), lambda i: (i, 0, 0)),
        out_shape=jax.ShapeDtypeStruct((T, N, N), jnp.float32),
    )(r_cm, x3, stats, gp, bp)


def kernel(X, edge_index, conv_w, conv_b, cheb_W, cheb_b, bn_gamma, bn_beta):
    x3 = X[0]
    src = edge_index[0].reshape(1, E)
    dst = edge_index[1].reshape(1, E)

    sabs, srcm, dstm = _prep(src, dst)
    srcm_w = srcm.reshape(32, E // 32 // EB, EB)
    dstm3 = dstm.reshape(NTILES, NBATCH, EB)
    sabs4 = sabs.reshape(NCHUNK, NTILES, NBATCH, EB)

    degp = _deg_kernel(srcm_w)
    y0_cm, u0_cm, dis = _conv(x3, conv_w[:, :, 0, 0], conv_b.reshape(1, N), degp)

    p1 = _spmm_kernel(u0_cm.reshape(NCHUNK * T, CW), sabs4, dstm3)
    u1 = _scale(p1.reshape(NCHUNK, T, CW), dis)
    p2 = _spmm_kernel(u1.reshape(NCHUNK * T, CW), sabs4, dstm3)

    cb2 = jnp.concatenate([cheb_b, cheb_b]).reshape(1, CW)
    r_cm, stats = _cheb(y0_cm, p1.reshape(NCHUNK, T, CW),
                        p2.reshape(NCHUNK, T, CW), dis, cheb_W, cb2)
    gp = jnp.repeat(bn_gamma[:, None], N, axis=1).reshape(NCHUNK, CW)
    bp = jnp.repeat(bn_beta[:, None], N, axis=1).reshape(NCHUNK, CW)
    out = _bn(r_cm, x3, stats, gp, bp)
    return out[None]


# speedup vs baseline: 10.5583x; 1.1162x over previous
import functools

import jax
import jax.numpy as jnp
from jax import lax
from jax.experimental import pallas as pl
from jax.experimental.pallas import tpu as pltpu
from jax.experimental.pallas import tpu_sc as plsc

T = 8192
N = 64
E = 131072
NCHUNK = 32
CW = 128
ACC_ROWS = 8448
ZROWS = 528
TRASH = 8192
NTILES = 16
EB = 128
NBATCH = E // NTILES // EB
TBLK = 128
CBLK = 256
NCBLK = T // CBLK
NTBLK = T // TBLK

_mesh = plsc.VectorSubcoreMesh(core_axis_name="c", subcore_axis_name="s")


def _prep_body(src_ref, dst_ref, sabs_ref, srcm_ref, dstm_ref):
    c = pl.program_id(0)
    s = src_ref[...]
    d = dst_ref[...]
    sabs_ref[...] = (s + c * T)[None]

    @pl.when(c == 0)
    def _():
        loop = s == d
        srcm_ref[...] = jnp.where(loop, TRASH, s)
        dstm_ref[...] = jnp.where(loop, TRASH, d)


def _prep(src, dst):
    return pl.pallas_call(
        _prep_body,
        grid=(NCHUNK,),
        in_specs=[
            pl.BlockSpec((1, E), lambda c: (0, 0)),
            pl.BlockSpec((1, E), lambda c: (0, 0)),
        ],
        out_specs=[
            pl.BlockSpec((1, 1, E), lambda c: (c, 0, 0)),
            pl.BlockSpec((1, E), lambda c: (0, 0)),
            pl.BlockSpec((1, E), lambda c: (0, 0)),
        ],
        out_shape=[
            jax.ShapeDtypeStruct((NCHUNK, 1, E), jnp.int32),
            jax.ShapeDtypeStruct((1, E), jnp.int32),
            jax.ShapeDtypeStruct((1, E), jnp.int32),
        ],
    )(src, dst)


@functools.partial(
    pl.kernel,
    out_type=jax.ShapeDtypeStruct((2, T), jnp.float32),
    mesh=_mesh,
    scratch_types=[
        pltpu.VMEM((E // 32 // EB, EB), jnp.int32),
        pltpu.VMEM((EB,), jnp.float32),
        pltpu.VMEM((ZROWS,), jnp.float32),
        pltpu.VMEM_SHARED((ACC_ROWS,), jnp.float32),
    ],
)
def _deg_kernel(srcm_hbm, degp_hbm, idx_v, ones_v, zero_v, acc_sh):
    cid = lax.axis_index("c")
    sid = lax.axis_index("s")
    wid = cid * NTILES + sid
    one = jnp.ones((16,), jnp.float32)
    zero = jnp.zeros((16,), jnp.float32)
    for k in range(EB // 16):
        ones_v[pl.ds(k * 16, 16)] = one
    for k in range(ZROWS // 16):
        zero_v[pl.ds(k * 16, 16)] = zero
    pltpu.sync_copy(zero_v, acc_sh.at[pl.ds(sid * ZROWS, ZROWS)])
    pltpu.sync_copy(srcm_hbm.at[wid], idx_v)
    plsc.subcore_barrier()

    def body(j, carry):
        pltpu.sync_copy(ones_v, acc_sh.at[idx_v.at[j]], add=True)
        return carry

    lax.fori_loop(0, E // 32 // EB, body, 0)
    plsc.subcore_barrier()
    pltpu.sync_copy(
        acc_sh.at[pl.ds(sid * (T // NTILES), T // NTILES)],
        degp_hbm.at[cid, pl.ds(sid * (T // NTILES), T // NTILES)],
    )


def _conv_body(x_ref, w_ref, b_ref, degp_ref, y0_ref, u0_ref, dis_ref):
    x = x_ref[...]
    w = w_ref[...]
    z = jnp.dot(x.reshape(TBLK * N, N), w.T, preferred_element_type=jnp.float32)
    z = jnp.maximum(z + b_ref[...], 0.0)
    y = jnp.swapaxes(z.reshape(TBLK, N, N), 1, 2)
    dg = degp_ref[...]
    deg = dg[0] + dg[1]
    dis = jnp.where(deg > 0.0, lax.rsqrt(jnp.maximum(deg, 1e-12)), 0.0)
    dis_ref[...] = dis[None, :]
    u = y * dis[:, None, None]
    for c in range(NCHUNK):
        y0_ref[c] = jnp.concatenate([y[:, 2 * c, :], y[:, 2 * c + 1, :]], axis=1)
        u0_ref[c] = jnp.concatenate([u[:, 2 * c, :], u[:, 2 * c + 1, :]], axis=1)


def _conv(x3, w, b, degp):
    return pl.pallas_call(
        _conv_body,
        grid=(NTBLK,),
        in_specs=[
            pl.BlockSpec((TBLK, N, N), lambda i: (i, 0, 0)),
            pl.BlockSpec((N, N), lambda i: (0, 0)),
            pl.BlockSpec((1, N), lambda i: (0, 0)),
            pl.BlockSpec((2, TBLK), lambda i: (0, i)),
        ],
        out_specs=[
            pl.BlockSpec((NCHUNK, TBLK, CW), lambda i: (0, i, 0)),
            pl.BlockSpec((NCHUNK, TBLK, CW), lambda i: (0, i, 0)),
            pl.BlockSpec((1, TBLK), lambda i: (0, i)),
        ],
        out_shape=[
            jax.ShapeDtypeStruct((NCHUNK, T, CW), jnp.float32),
            jax.ShapeDtypeStruct((NCHUNK, T, CW), jnp.float32),
            jax.ShapeDtypeStruct((1, T), jnp.float32),
        ],
    )(x3, w, b, degp)


@functools.partial(
    pl.kernel,
    out_type=jax.ShapeDtypeStruct((NCHUNK * T, CW), jnp.float32),
    mesh=_mesh,
    scratch_types=[
        pltpu.VMEM((NBATCH, EB), jnp.int32),
        pltpu.VMEM((NBATCH, EB), jnp.int32),
        pltpu.VMEM((EB, CW), jnp.float32),
        pltpu.VMEM((EB, CW), jnp.float32),
        pltpu.VMEM((ZROWS // 6, CW), jnp.float32),
        pltpu.VMEM_SHARED((ACC_ROWS, CW), jnp.float32),
        pltpu.SemaphoreType.DMA,
    ],
)
def _spmm_kernel(u_hbm, sabs_hbm, dstm_hbm, p_hbm,
                 sidx_v, didx_v, rows_a, rows_b, zero_v, acc_sh, sem):
    cid = lax.axis_index("c")
    sid = lax.axis_index("s")
    zero = jnp.zeros((16,), jnp.float32)

    def zb(i, carry):
        for k in range(CW // 16):
            zero_v[i, pl.ds(k * 16, 16)] = zero
        return carry

    lax.fori_loop(0, ZROWS // 6, zb, 0)
    pltpu.sync_copy(dstm_hbm.at[sid], didx_v)

    def chunk_body(k, carry):
        c = cid * NTILES + k
        for q in range(6):
            pltpu.sync_copy(
                zero_v,
                acc_sh.at[pl.ds(sid * ZROWS + q * (ZROWS // 6), ZROWS // 6)],
            )
        pltpu.sync_copy(sabs_hbm.at[c, sid], sidx_v)
        plsc.subcore_barrier()

        pltpu.async_copy(u_hbm.at[sidx_v.at[0]], rows_a, sem)

        def batch_body(i, bcarry):
            ja = 2 * i
            jb = 2 * i + 1
            pltpu.make_async_copy(u_hbm.at[sidx_v.at[0]], rows_a, sem).wait()
            pltpu.async_copy(u_hbm.at[sidx_v.at[jb]], rows_b, sem)
            pltpu.sync_copy(rows_a, acc_sh.at[didx_v.at[ja]], add=True)
            pltpu.make_async_copy(u_hbm.at[sidx_v.at[0]], rows_b, sem).wait()
            nxt = jnp.minimum(jb + 1, NBATCH - 1)
            pltpu.async_copy(u_hbm.at[sidx_v.at[nxt]], rows_a, sem)
            pltpu.sync_copy(rows_b, acc_sh.at[didx_v.at[jb]], add=True)
            return bcarry

        lax.fori_loop(0, NBATCH // 2, batch_body, 0)
        pltpu.make_async_copy(u_hbm.at[sidx_v.at[0]], rows_a, sem).wait()
        plsc.subcore_barrier()
        pltpu.sync_copy(
            acc_sh.at[pl.ds(sid * (T // NTILES), T // NTILES)],
            p_hbm.at[pl.ds(c * T + sid * (T // NTILES), T // NTILES)],
        )
        plsc.subcore_barrier()
        return carry

    lax.fori_loop(0, NTILES, chunk_body, 0)


def _scale_body(p_ref, dis_ref, u_ref):
    d = dis_ref[...]
    u_ref[...] = -(d * d)[:, :, None] * p_ref[...]


def _scale(p_cm, dis):
    return pl.pallas_call(
        _scale_body,
        grid=(NCHUNK, NCBLK),
        in_specs=[
            pl.BlockSpec((1, CBLK, CW), lambda c, i: (c, i, 0)),
            pl.BlockSpec((1, CBLK), lambda c, i: (0, i)),
        ],
        out_specs=pl.BlockSpec((1, CBLK, CW), lambda c, i: (c, i, 0)),
        out_shape=jax.ShapeDtypeStruct((NCHUNK, T, CW), jnp.float32),
    )(p_cm, dis)


def _bd(w):
    zz = jnp.zeros((N, N), jnp.float32)
    top = jnp.concatenate([w, zz], axis=1)
    bot = jnp.concatenate([zz, w], axis=1)
    return jnp.concatenate([top, bot], axis=0)


def _cheb_body(y0_ref, p1_ref, p2_ref, dis_ref, w_ref, b_ref, r_ref, st_ref):
    y0 = y0_ref[...].reshape(NCHUNK * CBLK, CW)
    p1 = p1_ref[...].reshape(NCHUNK * CBLK, CW)
    p2 = p2_ref[...].reshape(NCHUNK * CBLK, CW)
    w = w_ref[...]
    m1 = jnp.dot(y0, _bd(w[0] - w[2]), preferred_element_type=jnp.float32)
    m2 = (jnp.dot(p1, _bd(w[1]), preferred_element_type=jnp.float32)
          + jnp.dot(p2, _bd(2.0 * w[2]), preferred_element_type=jnp.float32))
    d = dis_ref[...][0]
    r = (m1.reshape(NCHUNK, CBLK, CW)
         - d[None, :, None] * m2.reshape(NCHUNK, CBLK, CW)
         + b_ref[...][0][None, None, :])
    r_ref[...] = r
    s1 = r.sum(axis=1)
    s2 = (r * r).sum(axis=1)
    st_ref[...] = jnp.concatenate([s1, s2], axis=0)[None]


def _cheb(y0_cm, p1_cm, p2_cm, dis, cheb_W, cheb_b2):
    return pl.pallas_call(
        _cheb_body,
        grid=(NCBLK,),
        in_specs=[
            pl.BlockSpec((NCHUNK, CBLK, CW), lambda i: (0, i, 0)),
            pl.BlockSpec((NCHUNK, CBLK, CW), lambda i: (0, i, 0)),
            pl.BlockSpec((NCHUNK, CBLK, CW), lambda i: (0, i, 0)),
            pl.BlockSpec((1, CBLK), lambda i: (0, i)),
            pl.BlockSpec((3, N, N), lambda i: (0, 0, 0)),
            pl.BlockSpec((1, CW), lambda i: (0, 0)),
        ],
        out_specs=[
            pl.BlockSpec((NCHUNK, CBLK, CW), lambda i: (0, i, 0)),
            pl.BlockSpec((1, 2 * NCHUNK, CW), lambda i: (i, 0, 0)),
        ],
        out_shape=[
            jax.ShapeDtypeStruct((NCHUNK, T, CW), jnp.float32),
            jax.ShapeDtypeStruct((NCBLK, 2 * NCHUNK, CW), jnp.float32),
        ],
    )(y0_cm, p1_cm, p2_cm, dis, cheb_W, cheb_b2)


def _bn_body(r_ref, x_ref, st_ref, g_ref, be_ref, out_ref):
    st = st_ref[...].sum(axis=0)
    cnt = float(T * N)
    s1l = st[:NCHUNK, : N].sum(axis=1)[:, None]
    s1r = st[:NCHUNK, N:].sum(axis=1)[:, None]
    s2l = st[NCHUNK:, : N].sum(axis=1)[:, None]
    s2r = st[NCHUNK:, N:].sum(axis=1)[:, None]
    ones = jnp.ones((1, N), jnp.float32)
    mu = jnp.concatenate([s1l * ones, s1r * ones], axis=1) / cnt
    s2 = jnp.concatenate([s2l * ones, s2r * ones], axis=1) / cnt
    var = s2 - mu * mu
    scale = g_ref[...] * lax.rsqrt(var + 1e-5)
    shift = be_ref[...] - mu * scale
    q = r_ref[...] * scale[:, None, :] + shift[:, None, :]
    x = x_ref[...]
    for c in range(NCHUNK):
        out_ref[:, 2 * c, :] = jnp.maximum(q[c, :, :N] + x[:, 2 * c, :], 0.0)
        out_ref[:, 2 * c + 1, :] = jnp.maximum(q[c, :, N:] + x[:, 2 * c + 1, :], 0.0)


def _bn(r_cm, x3, stats, gp, bp):
    return pl.pallas_call(
        _bn_body,
        grid=(NCBLK,),
        in_specs=[
            pl.BlockSpec((NCHUNK, CBLK, CW), lambda i: (0, i, 0)),
            pl.BlockSpec((CBLK, N, N), lambda i: (i, 0, 0)),
            pl.BlockSpec((NCBLK, 2 * NCHUNK, CW), lambda i: (0, 0, 0)),
            pl.BlockSpec((NCHUNK, CW), lambda i: (0, 0)),
            pl.BlockSpec((NCHUNK, CW), lambda i: (0, 0)),
        ],
        out_specs=pl.BlockSpec((CBLK, N, N), lambda i: (i, 0, 0)),
        out_shape=jax.ShapeDtypeStruct((T, N, N), jnp.float32),
    )(r_cm, x3, stats, gp, bp)


def kernel(X, edge_index, conv_w, conv_b, cheb_W, cheb_b, bn_gamma, bn_beta):
    x3 = X[0]
    src = edge_index[0].reshape(1, E)
    dst = edge_index[1].reshape(1, E)

    sabs, srcm, dstm = _prep(src, dst)
    srcm_w = srcm.reshape(32, E // 32 // EB, EB)
    dstm3 = dstm.reshape(NTILES, NBATCH, EB)
    sabs4 = sabs.reshape(NCHUNK, NTILES, NBATCH, EB)

    degp = _deg_kernel(srcm_w)
    y0_cm, u0_cm, dis = _conv(x3, conv_w[:, :, 0, 0], conv_b.reshape(1, N), degp)

    p1 = _spmm_kernel(u0_cm.reshape(NCHUNK * T, CW), sabs4, dstm3)
    u1 = _scale(p1.reshape(NCHUNK, T, CW), dis)
    p2 = _spmm_kernel(u1.reshape(NCHUNK * T, CW), sabs4, dstm3)

    cb2 = jnp.concatenate([cheb_b, cheb_b]).reshape(1, CW)
    r_cm, stats = _cheb(y0_cm, p1.reshape(NCHUNK, T, CW),
                        p2.reshape(NCHUNK, T, CW), dis, cheb_W, cb2)
    gp = jnp.repeat(bn_gamma[:, None], N, axis=1).reshape(NCHUNK, CW)
    bp = jnp.repeat(bn_beta[:, None], N, axis=1).reshape(NCHUNK, CW)
    out = _bn(r_cm, x3, stats, gp, bp)
    return out[None]
